# baseline (speedup 1.0000x reference)
import functools

import jax
import jax.numpy as jnp
from jax import lax
from jax.experimental import pallas as pl
from jax.experimental.pallas import tpu as pltpu
from jax.experimental.pallas import tpu_sc as plsc

K = 2
NC, NS = 2, 16
NW = NC * NS
BR = 512
BM = 256
BD = 512

_INTERPRET = False


def _router_body(x_ref, wg_ref, e0_ref, e1_ref, w0_ref, w1_ref, stats_ref):
    E = wg_ref.shape[-1]
    x = x_ref[...]
    logits = jnp.dot(x, wg_ref[...], preferred_element_type=jnp.float32)
    m = jnp.max(logits, axis=-1, keepdims=True)
    ex = jnp.exp(logits - m)
    s = jnp.sum(ex, axis=-1, keepdims=True)
    probs = ex / s
    lse = m[:, 0] + jnp.log(s[:, 0])

    eidx = lax.broadcasted_iota(jnp.int32, probs.shape, 1)
    p0 = jnp.max(probs, axis=-1)
    is0 = probs == p0[:, None]
    i0 = jnp.min(jnp.where(is0, eidx, E), axis=-1).astype(jnp.int32)
    probs_m = jnp.where(eidx == i0[:, None], -1.0, probs)
    p1 = jnp.max(probs_m, axis=-1)
    is1 = probs_m == p1[:, None]
    i1 = jnp.min(jnp.where(is1, eidx, E), axis=-1).astype(jnp.int32)
    tot = p0 + p1
    e0_ref[...] = i0
    e1_ref[...] = i1
    w0_ref[...] = p0 / tot
    w1_ref[...] = p1 / tot

    onehot = (eidx == i0[:, None]).astype(jnp.float32) + (
        eidx == i1[:, None]
    ).astype(jnp.float32)
    psum = jnp.sum(probs, axis=0)
    csum = jnp.sum(onehot, axis=0)
    zsum = jnp.sum(lse * lse)
    vec = jnp.concatenate(
        [psum, csum, jnp.full((1,), zsum, jnp.float32),
         jnp.zeros((128 - 2 * E - 1,), jnp.float32)]
    )
    stats_ref[0, 0, :] = vec


def _router(xf, wgT):
    T, H = xf.shape
    E = wgT.shape[1]
    nblk = T // BR
    return pl.pallas_call(
        _router_body,
        grid=(nblk,),
        in_specs=[
            pl.BlockSpec((BR, H), lambda i: (i, 0)),
            pl.BlockSpec((H, E), lambda i: (0, 0)),
        ],
        out_specs=[
            pl.BlockSpec((BR,), lambda i: (i,)),
            pl.BlockSpec((BR,), lambda i: (i,)),
            pl.BlockSpec((BR,), lambda i: (i,)),
            pl.BlockSpec((BR,), lambda i: (i,)),
            pl.BlockSpec((1, 1, 128), lambda i: (i, 0, 0)),
        ],
        out_shape=[
            jax.ShapeDtypeStruct((T,), jnp.int32),
            jax.ShapeDtypeStruct((T,), jnp.int32),
            jax.ShapeDtypeStruct((T,), jnp.float32),
            jax.ShapeDtypeStruct((T,), jnp.float32),
            jax.ShapeDtypeStruct((nblk, 1, 128), jnp.float32),
        ],
        interpret=_INTERPRET,
    )(xf, wgT)


def _dest_body(e0_ref, e1_ref, ps_ref, d0_ref, d1_ref, run_ref):
    E = ps_ref.shape[-1]
    i = pl.program_id(0)

    @pl.when(i == 0)
    def _():
        run_ref[...] = jnp.zeros_like(run_ref)

    e01 = jnp.concatenate([e0_ref[...], e1_ref[...]])
    eidx = lax.broadcasted_iota(jnp.int32, (2 * BD, E), 1)
    onehot = (e01[:, None] == eidx).astype(jnp.float32)
    r = lax.broadcasted_iota(jnp.int32, (2 * BD, 2 * BD), 0)
    c = lax.broadcasted_iota(jnp.int32, (2 * BD, 2 * BD), 1)
    tril = (r > c).astype(jnp.float32)
    m = jnp.dot(tril, onehot, preferred_element_type=jnp.float32)
    intra = jnp.sum(onehot * m, axis=1)
    base = ps_ref[0, :] + run_ref[0, :]
    destp = jnp.sum(onehot * base[None, :], axis=1) + intra
    d0_ref[...] = destp[:BD].astype(jnp.int32)
    d1_ref[...] = destp[BD:].astype(jnp.int32)
    run_ref[0, :] += jnp.sum(onehot, axis=0)


def _dest(e0, e1, pstart):
    T = e0.shape[0]
    E = pstart.shape[-1]
    nblk = T // BD
    return pl.pallas_call(
        _dest_body,
        grid=(nblk,),
        in_specs=[
            pl.BlockSpec((BD,), lambda i: (i,)),
            pl.BlockSpec((BD,), lambda i: (i,)),
            pl.BlockSpec((1, E), lambda i: (0, 0)),
        ],
        out_specs=[
            pl.BlockSpec((BD,), lambda i: (i,)),
            pl.BlockSpec((BD,), lambda i: (i,)),
        ],
        out_shape=[
            jax.ShapeDtypeStruct((T,), jnp.int32),
            jax.ShapeDtypeStruct((T,), jnp.int32),
        ],
        scratch_shapes=[pltpu.VMEM((1, E), jnp.float32)],
        interpret=_INTERPRET,
    )(e0, e1, pstart)


def _ffn_body(be_ref, x_ref, w1_ref, w3_ref, w2_ref, y_ref):
    x = x_ref[...]
    a = jnp.dot(x, w1_ref[0], preferred_element_type=jnp.float32)
    b = jnp.dot(x, w3_ref[0], preferred_element_type=jnp.float32)
    h = a * jax.nn.sigmoid(a) * b
    y_ref[...] = jnp.dot(h, w2_ref[0], preferred_element_type=jnp.float32)


def _ffn(block_expert, x_pad, w1, w3, w2):
    PT, H = x_pad.shape
    E, _, F = w1.shape
    nb = PT // BM
    grid_spec = pltpu.PrefetchScalarGridSpec(
        num_scalar_prefetch=1,
        grid=(nb,),
        in_specs=[
            pl.BlockSpec((BM, H), lambda i, be: (i, 0)),
            pl.BlockSpec((1, H, F), lambda i, be: (be[i], 0, 0)),
            pl.BlockSpec((1, H, F), lambda i, be: (be[i], 0, 0)),
            pl.BlockSpec((1, F, H), lambda i, be: (be[i], 0, 0)),
        ],
        out_specs=pl.BlockSpec((BM, H), lambda i, be: (i, 0)),
    )
    return pl.pallas_call(
        _ffn_body,
        grid_spec=grid_spec,
        out_shape=jax.ShapeDtypeStruct((PT, H), jnp.float32),
        compiler_params=pltpu.CompilerParams(
            dimension_semantics=("arbitrary",),
            vmem_limit_bytes=100 * 1024 * 1024,
        ),
        interpret=_INTERPRET,
    )(block_expert, x_pad, w1, w3, w2)


def _sc_dispatch(xf, d0, d1, PT, CH=64):
    T, H = xf.shape
    per_w = T // NW
    nch = per_w // CH
    mesh = plsc.VectorSubcoreMesh(core_axis_name="c", subcore_axis_name="s")

    @functools.partial(
        pl.kernel,
        mesh=mesh,
        out_type=jax.ShapeDtypeStruct((PT, H), jnp.float32),
        scratch_types=[
            pltpu.VMEM((nch, CH), jnp.int32),
            pltpu.VMEM((nch, CH), jnp.int32),
            pltpu.VMEM((CH, H), jnp.float32),
            pltpu.SemaphoreType.DMA,
        ],
    )
    def k(x_hbm, d0_hbm, d1_hbm, out_hbm, d0_v, d1_v, rows_v, sem):
        wid = lax.axis_index("s") * NC + lax.axis_index("c")
        tbase = wid * per_w

        def body(c, carry):
            tb = tbase + c * CH
            pltpu.sync_copy(d0_hbm.at[pl.ds(tb, CH)], d0_v.at[c])
            pltpu.sync_copy(d1_hbm.at[pl.ds(tb, CH)], d1_v.at[c])
            pltpu.sync_copy(x_hbm.at[pl.ds(tb, CH)], rows_v)
            pltpu.async_copy(rows_v, out_hbm.at[d0_v.at[c]], sem)
            pltpu.async_copy(rows_v, out_hbm.at[d1_v.at[c]], sem)
            pltpu.make_async_copy(rows_v, out_hbm.at[d0_v.at[c]], sem).wait()
            pltpu.make_async_copy(rows_v, out_hbm.at[d1_v.at[c]], sem).wait()
            return carry

        lax.fori_loop(0, nch, body, 0)

    return k(xf, d0, d1)


def _lane_bcast(v16, j):
    idx = (jnp.zeros((16, 1), jnp.int32) + j).astype(jnp.int32)
    return lax.gather(
        v16, idx,
        lax.GatherDimensionNumbers(
            offset_dims=(), collapsed_slice_dims=(0,), start_index_map=(0,)
        ),
        slice_sizes=(1,),
        mode=lax.GatherScatterMode.PROMISE_IN_BOUNDS,
    )


def _sc_combine(y, d0, d1, w0, w1, CH=16):
    T = d0.shape[0]
    H = y.shape[1]
    per_w = T // NW
    nch = per_w // CH
    mesh = plsc.VectorSubcoreMesh(core_axis_name="c", subcore_axis_name="s")

    @functools.partial(
        pl.kernel,
        mesh=mesh,
        out_type=jax.ShapeDtypeStruct((T, H), jnp.float32),
        scratch_types=[
            pltpu.VMEM((per_w,), jnp.int32),
            pltpu.VMEM((per_w,), jnp.int32),
            pltpu.VMEM((per_w,), jnp.float32),
            pltpu.VMEM((per_w,), jnp.float32),
            [pltpu.VMEM((CH, H), jnp.float32) for _ in range(2)],
            [pltpu.VMEM((CH, H), jnp.float32) for _ in range(2)],
            [pltpu.SemaphoreType.DMA for _ in range(2)],
        ],
    )
    def k(y_hbm, d0_hbm, d1_hbm, w0_hbm, w1_hbm, out_hbm,
          i0_v, i1_v, w0_v, w1_v, r0_v, r1_v, sems):
        wid = lax.axis_index("s") * NC + lax.axis_index("c")
        base = wid * per_w
        pltpu.sync_copy(d0_hbm.at[pl.ds(base, per_w)], i0_v)
        pltpu.sync_copy(d1_hbm.at[pl.ds(base, per_w)], i1_v)
        pltpu.sync_copy(w0_hbm.at[pl.ds(base, per_w)], w0_v)
        pltpu.sync_copy(w1_hbm.at[pl.ds(base, per_w)], w1_v)

        def launch(c, sl):
            cs = pl.ds(c * CH, CH)
            pltpu.async_copy(y_hbm.at[i0_v.at[cs]], r0_v[sl], sems[sl])
            pltpu.async_copy(y_hbm.at[i1_v.at[cs]], r1_v[sl], sems[sl])

        launch(0, 0)
        for c in range(nch):
            sl = c & 1
            if c + 1 < nch:
                launch(c + 1, sl ^ 1)
            cs = pl.ds(c * CH, CH)
            pltpu.make_async_copy(y_hbm.at[i0_v.at[cs]], r0_v[sl], sems[sl]).wait()
            pltpu.make_async_copy(y_hbm.at[i1_v.at[cs]], r1_v[sl], sems[sl]).wait()
            wq0 = w0_v[pl.ds(c * CH, 16)]
            wq1 = w1_v[pl.ds(c * CH, 16)]

            def wrow(r16, cc, sl=sl, wq0=wq0, wq1=wq1):
                b0 = _lane_bcast(wq0, r16)
                b1 = _lane_bcast(wq1, r16)
                for col in range(H // 16):
                    csl = pl.ds(col * 16, 16)
                    r0_v[sl][r16, csl] = (
                        b0 * r0_v[sl][r16, csl] + b1 * r1_v[sl][r16, csl]
                    )
                return cc

            lax.fori_loop(0, CH, wrow, 0)
            pltpu.sync_copy(r0_v[sl], out_hbm.at[pl.ds(base + c * CH, CH)])

    return k(y, d0, d1, w0, w1)


def kernel(x, W_gate, w1, w3, w2):
    b, s, H = x.shape
    T = b * s
    E = W_gate.shape[0]
    xf = x.reshape(T, H)

    e0, e1, w0v, w1v, stats = _router(xf, W_gate.T)
    ssum = jnp.sum(stats, axis=(0, 1))
    probs_sum = ssum[:E]
    counts = ssum[E:2 * E]
    zsum = ssum[2 * E]
    balance_loss = E * jnp.sum((counts / (T * K)) * (probs_sum / T))
    z_loss = zsum / T

    PT = T * K + E * BM
    g = counts.astype(jnp.int32)
    padded = ((g + BM - 1) // BM) * BM
    pend = jnp.cumsum(padded).astype(jnp.int32)
    pstart = pend - padded
    d0, d1 = _dest(e0, e1, pstart.astype(jnp.float32)[None, :])
    nb = PT // BM
    bstart = jnp.arange(nb, dtype=jnp.int32) * BM
    block_expert = jnp.minimum(
        jnp.sum((bstart[:, None] >= pend[None, :]).astype(jnp.int32), axis=1),
        E - 1,
    ).astype(jnp.int32)

    x_pad = _sc_dispatch(xf, d0, d1, PT)

    y = _ffn(block_expert, x_pad, w1, w3, w2)

    out = _sc_combine(y, d0, d1, w0v, w1v)

    return out.reshape(b, s, H), balance_loss, z_loss


# speedup vs baseline: 1.0323x; 1.0323x over previous
import functools

import jax
import jax.numpy as jnp
from jax import lax
from jax.experimental import pallas as pl
from jax.experimental.pallas import tpu as pltpu
from jax.experimental.pallas import tpu_sc as plsc

K = 2
NC, NS = 2, 16
NW = NC * NS
BR = 512
BM = 256

_INTERPRET = False


def _router_body(x_ref, wg_ref, e0_ref, e1_ref, w0_ref, w1_ref, stats_ref):
    E = wg_ref.shape[-1]
    x = x_ref[...]
    logits = jnp.dot(x, wg_ref[...], preferred_element_type=jnp.float32)
    m = jnp.max(logits, axis=-1, keepdims=True)
    ex = jnp.exp(logits - m)
    s = jnp.sum(ex, axis=-1, keepdims=True)
    probs = ex / s
    lse = m[:, 0] + jnp.log(s[:, 0])

    eidx = lax.broadcasted_iota(jnp.int32, probs.shape, 1)
    p0 = jnp.max(probs, axis=-1)
    is0 = probs == p0[:, None]
    i0 = jnp.min(jnp.where(is0, eidx, E), axis=-1).astype(jnp.int32)
    probs_m = jnp.where(eidx == i0[:, None], -1.0, probs)
    p1 = jnp.max(probs_m, axis=-1)
    is1 = probs_m == p1[:, None]
    i1 = jnp.min(jnp.where(is1, eidx, E), axis=-1).astype(jnp.int32)
    tot = p0 + p1
    e0_ref[...] = i0
    e1_ref[...] = i1
    w0_ref[...] = p0 / tot
    w1_ref[...] = p1 / tot

    onehot = (eidx == i0[:, None]).astype(jnp.float32) + (
        eidx == i1[:, None]
    ).astype(jnp.float32)
    psum = jnp.sum(probs, axis=0)
    csum = jnp.sum(onehot, axis=0)
    zsum = jnp.sum(lse * lse)
    vec = jnp.concatenate(
        [psum, csum, jnp.full((1,), zsum, jnp.float32),
         jnp.zeros((128 - 2 * E - 1,), jnp.float32)]
    )
    stats_ref[0, 0, :] = vec


def _router(xf, wgT):
    T, H = xf.shape
    E = wgT.shape[1]
    nblk = T // BR
    return pl.pallas_call(
        _router_body,
        grid=(nblk,),
        in_specs=[
            pl.BlockSpec((BR, H), lambda i: (i, 0)),
            pl.BlockSpec((H, E), lambda i: (0, 0)),
        ],
        out_specs=[
            pl.BlockSpec((BR,), lambda i: (i,)),
            pl.BlockSpec((BR,), lambda i: (i,)),
            pl.BlockSpec((BR,), lambda i: (i,)),
            pl.BlockSpec((BR,), lambda i: (i,)),
            pl.BlockSpec((1, 1, 128), lambda i: (i, 0, 0)),
        ],
        out_shape=[
            jax.ShapeDtypeStruct((T,), jnp.int32),
            jax.ShapeDtypeStruct((T,), jnp.int32),
            jax.ShapeDtypeStruct((T,), jnp.float32),
            jax.ShapeDtypeStruct((T,), jnp.float32),
            jax.ShapeDtypeStruct((nblk, 1, 128), jnp.float32),
        ],
        interpret=_INTERPRET,
    )(xf, wgT)


def _ffn_body(be_ref, x_ref, w1_ref, w3_ref, w2_ref, y_ref):
    x = x_ref[...]
    a = jnp.dot(x, w1_ref[0], preferred_element_type=jnp.float32)
    b = jnp.dot(x, w3_ref[0], preferred_element_type=jnp.float32)
    h = a * jax.nn.sigmoid(a) * b
    y_ref[...] = jnp.dot(h, w2_ref[0], preferred_element_type=jnp.float32)


def _ffn(block_expert, x_pad, w1, w3, w2):
    PT, H = x_pad.shape
    E, _, F = w1.shape
    nb = PT // BM
    grid_spec = pltpu.PrefetchScalarGridSpec(
        num_scalar_prefetch=1,
        grid=(nb,),
        in_specs=[
            pl.BlockSpec((BM, H), lambda i, be: (i, 0)),
            pl.BlockSpec((1, H, F), lambda i, be: (be[i], 0, 0)),
            pl.BlockSpec((1, H, F), lambda i, be: (be[i], 0, 0)),
            pl.BlockSpec((1, F, H), lambda i, be: (be[i], 0, 0)),
        ],
        out_specs=pl.BlockSpec((BM, H), lambda i, be: (i, 0)),
    )
    return pl.pallas_call(
        _ffn_body,
        grid_spec=grid_spec,
        out_shape=jax.ShapeDtypeStruct((PT, H), jnp.float32),
        compiler_params=pltpu.CompilerParams(
            dimension_semantics=("arbitrary",),
            vmem_limit_bytes=100 * 1024 * 1024,
        ),
        interpret=_INTERPRET,
    )(block_expert, x_pad, w1, w3, w2)


def _sc_dispatch(xf, d0, d1, PT, CH=64):
    T, H = xf.shape
    per_w = T // NW
    nch = per_w // CH
    mesh = plsc.VectorSubcoreMesh(core_axis_name="c", subcore_axis_name="s")

    @functools.partial(
        pl.kernel,
        mesh=mesh,
        out_type=jax.ShapeDtypeStruct((PT, H), jnp.float32),
        scratch_types=[
            pltpu.VMEM((nch, CH), jnp.int32),
            pltpu.VMEM((nch, CH), jnp.int32),
            pltpu.VMEM((CH, H), jnp.float32),
            pltpu.SemaphoreType.DMA,
        ],
    )
    def k(x_hbm, d0_hbm, d1_hbm, out_hbm, d0_v, d1_v, rows_v, sem):
        wid = lax.axis_index("s") * NC + lax.axis_index("c")
        tbase = wid * per_w

        def body(c, carry):
            tb = tbase + c * CH
            pltpu.sync_copy(d0_hbm.at[pl.ds(tb, CH)], d0_v.at[c])
            pltpu.sync_copy(d1_hbm.at[pl.ds(tb, CH)], d1_v.at[c])
            pltpu.sync_copy(x_hbm.at[pl.ds(tb, CH)], rows_v)
            pltpu.async_copy(rows_v, out_hbm.at[d0_v.at[c]], sem)
            pltpu.async_copy(rows_v, out_hbm.at[d1_v.at[c]], sem)
            pltpu.make_async_copy(rows_v, out_hbm.at[d0_v.at[c]], sem).wait()
            pltpu.make_async_copy(rows_v, out_hbm.at[d1_v.at[c]], sem).wait()
            return carry

        lax.fori_loop(0, nch, body, 0)

    return k(xf, d0, d1)


def _lane_bcast(v16, j):
    idx = (jnp.zeros((16, 1), jnp.int32) + j).astype(jnp.int32)
    return lax.gather(
        v16, idx,
        lax.GatherDimensionNumbers(
            offset_dims=(), collapsed_slice_dims=(0,), start_index_map=(0,)
        ),
        slice_sizes=(1,),
        mode=lax.GatherScatterMode.PROMISE_IN_BOUNDS,
    )


def _sc_combine(y, d0, d1, w0, w1, CH=16):
    T = d0.shape[0]
    H = y.shape[1]
    per_w = T // NW
    nch = per_w // CH
    mesh = plsc.VectorSubcoreMesh(core_axis_name="c", subcore_axis_name="s")

    @functools.partial(
        pl.kernel,
        mesh=mesh,
        out_type=jax.ShapeDtypeStruct((T, H), jnp.float32),
        scratch_types=[
            pltpu.VMEM((per_w,), jnp.int32),
            pltpu.VMEM((per_w,), jnp.int32),
            pltpu.VMEM((per_w,), jnp.float32),
            pltpu.VMEM((per_w,), jnp.float32),
            [pltpu.VMEM((CH, H), jnp.float32) for _ in range(2)],
            [pltpu.VMEM((CH, H), jnp.float32) for _ in range(2)],
            [pltpu.SemaphoreType.DMA for _ in range(2)],
        ],
    )
    def k(y_hbm, d0_hbm, d1_hbm, w0_hbm, w1_hbm, out_hbm,
          i0_v, i1_v, w0_v, w1_v, r0_v, r1_v, sems):
        wid = lax.axis_index("s") * NC + lax.axis_index("c")
        base = wid * per_w
        pltpu.sync_copy(d0_hbm.at[pl.ds(base, per_w)], i0_v)
        pltpu.sync_copy(d1_hbm.at[pl.ds(base, per_w)], i1_v)
        pltpu.sync_copy(w0_hbm.at[pl.ds(base, per_w)], w0_v)
        pltpu.sync_copy(w1_hbm.at[pl.ds(base, per_w)], w1_v)

        def launch(c, sl):
            cs = pl.ds(c * CH, CH)
            pltpu.async_copy(y_hbm.at[i0_v.at[cs]], r0_v[sl], sems[sl])
            pltpu.async_copy(y_hbm.at[i1_v.at[cs]], r1_v[sl], sems[sl])

        launch(0, 0)
        for c in range(nch):
            sl = c & 1
            if c + 1 < nch:
                launch(c + 1, sl ^ 1)
            cs = pl.ds(c * CH, CH)
            pltpu.make_async_copy(y_hbm.at[i0_v.at[cs]], r0_v[sl], sems[sl]).wait()
            pltpu.make_async_copy(y_hbm.at[i1_v.at[cs]], r1_v[sl], sems[sl]).wait()
            wq0 = w0_v[pl.ds(c * CH, 16)]
            wq1 = w1_v[pl.ds(c * CH, 16)]

            def wrow(r16, cc, sl=sl, wq0=wq0, wq1=wq1):
                b0 = _lane_bcast(wq0, r16)
                b1 = _lane_bcast(wq1, r16)
                for col in range(H // 16):
                    csl = pl.ds(col * 16, 16)
                    r0_v[sl][r16, csl] = (
                        b0 * r0_v[sl][r16, csl] + b1 * r1_v[sl][r16, csl]
                    )
                return cc

            lax.fori_loop(0, CH, wrow, 0)
            pltpu.sync_copy(r0_v[sl], out_hbm.at[pl.ds(base + c * CH, CH)])

    return k(y, d0, d1, w0, w1)


def kernel(x, W_gate, w1, w3, w2):
    b, s, H = x.shape
    T = b * s
    E = W_gate.shape[0]
    xf = x.reshape(T, H)

    e0, e1, w0v, w1v, stats = _router(xf, W_gate.T)
    ssum = jnp.sum(stats, axis=(0, 1))
    probs_sum = ssum[:E]
    counts = ssum[E:2 * E]
    zsum = ssum[2 * E]
    balance_loss = E * jnp.sum((counts / (T * K)) * (probs_sum / T))
    z_loss = zsum / T

    PT = T * K + E * BM
    e_flat = jnp.concatenate([e0, e1])
    onehot = (e_flat[:, None] == jnp.arange(E)[None, :]).astype(jnp.int32)
    csum = jnp.cumsum(onehot, axis=0)
    rank = jnp.sum((csum - 1) * onehot, axis=1)
    g = csum[-1]
    padded = ((g + BM - 1) // BM) * BM
    pend = jnp.cumsum(padded).astype(jnp.int32)
    pstart = pend - padded.astype(jnp.int32)
    dest = jnp.sum(onehot * pstart[None, :], axis=1) + rank
    nb = PT // BM
    bstart = jnp.arange(nb, dtype=jnp.int32) * BM
    block_expert = jnp.minimum(
        jnp.sum((bstart[:, None] >= pend[None, :]).astype(jnp.int32), axis=1),
        E - 1,
    ).astype(jnp.int32)
    d0 = dest[:T]
    d1 = dest[T:]

    x_pad = _sc_dispatch(xf, d0, d1, PT)

    y = _ffn(block_expert, x_pad, w1, w3, w2)

    out = _sc_combine(y, d0, d1, w0v, w1v)

    return out.reshape(b, s, H), balance_loss, z_loss


# speedup vs baseline: 1.0337x; 1.0013x over previous
import functools

import jax
import jax.numpy as jnp
from jax import lax
from jax.experimental import pallas as pl
from jax.experimental.pallas import tpu as pltpu
from jax.experimental.pallas import tpu_sc as plsc

K = 2
NC, NS = 2, 16
NW = NC * NS
BR = 512
BM = 256


def _router_body(x_ref, wg_ref, e0_ref, e1_ref, w0_ref, w1_ref, stats_ref):
    E = wg_ref.shape[-1]
    x = x_ref[...]
    logits = jnp.dot(x, wg_ref[...], preferred_element_type=jnp.float32)
    m = jnp.max(logits, axis=-1, keepdims=True)
    ex = jnp.exp(logits - m)
    s = jnp.sum(ex, axis=-1, keepdims=True)
    probs = ex / s
    lse = m[:, 0] + jnp.log(s[:, 0])

    eidx = lax.broadcasted_iota(jnp.int32, probs.shape, 1)
    p0 = jnp.max(probs, axis=-1)
    is0 = probs == p0[:, None]
    i0 = jnp.min(jnp.where(is0, eidx, E), axis=-1).astype(jnp.int32)
    probs_m = jnp.where(eidx == i0[:, None], -1.0, probs)
    p1 = jnp.max(probs_m, axis=-1)
    is1 = probs_m == p1[:, None]
    i1 = jnp.min(jnp.where(is1, eidx, E), axis=-1).astype(jnp.int32)
    tot = p0 + p1
    e0_ref[...] = i0
    e1_ref[...] = i1
    w0_ref[...] = p0 / tot
    w1_ref[...] = p1 / tot

    onehot = (eidx == i0[:, None]).astype(jnp.float32) + (
        eidx == i1[:, None]
    ).astype(jnp.float32)
    psum = jnp.sum(probs, axis=0)
    csum = jnp.sum(onehot, axis=0)
    zsum = jnp.sum(lse * lse)
    vec = jnp.concatenate(
        [psum, csum, jnp.full((1,), zsum, jnp.float32),
         jnp.zeros((128 - 2 * E - 1,), jnp.float32)]
    )
    stats_ref[0, 0, :] = vec


def _router(xf, wgT):
    T, H = xf.shape
    E = wgT.shape[1]
    nblk = T // BR
    return pl.pallas_call(
        _router_body,
        grid=(nblk,),
        in_specs=[
            pl.BlockSpec((BR, H), lambda i: (i, 0)),
            pl.BlockSpec((H, E), lambda i: (0, 0)),
        ],
        out_specs=[
            pl.BlockSpec((BR,), lambda i: (i,)),
            pl.BlockSpec((BR,), lambda i: (i,)),
            pl.BlockSpec((BR,), lambda i: (i,)),
            pl.BlockSpec((BR,), lambda i: (i,)),
            pl.BlockSpec((1, 1, 128), lambda i: (i, 0, 0)),
        ],
        out_shape=[
            jax.ShapeDtypeStruct((T,), jnp.int32),
            jax.ShapeDtypeStruct((T,), jnp.int32),
            jax.ShapeDtypeStruct((T,), jnp.float32),
            jax.ShapeDtypeStruct((T,), jnp.float32),
            jax.ShapeDtypeStruct((nblk, 1, 128), jnp.float32),
        ],
    )(xf, wgT)


def _ffn_body(be_ref, x_ref, w1_ref, w3_ref, w2_ref, y_ref):
    x = x_ref[...]
    a = jnp.dot(x, w1_ref[0], preferred_element_type=jnp.float32)
    b = jnp.dot(x, w3_ref[0], preferred_element_type=jnp.float32)
    h = a * jax.nn.sigmoid(a) * b
    y_ref[...] = jnp.dot(h, w2_ref[0], preferred_element_type=jnp.float32)


def _ffn(block_expert, x_pad, w1, w3, w2):
    PT, H = x_pad.shape
    E, _, F = w1.shape
    nb = PT // BM
    grid_spec = pltpu.PrefetchScalarGridSpec(
        num_scalar_prefetch=1,
        grid=(nb,),
        in_specs=[
            pl.BlockSpec((BM, H), lambda i, be: (i, 0)),
            pl.BlockSpec((1, H, F), lambda i, be: (be[i], 0, 0)),
            pl.BlockSpec((1, H, F), lambda i, be: (be[i], 0, 0)),
            pl.BlockSpec((1, F, H), lambda i, be: (be[i], 0, 0)),
        ],
        out_specs=pl.BlockSpec((BM, H), lambda i, be: (i, 0)),
    )
    return pl.pallas_call(
        _ffn_body,
        grid_spec=grid_spec,
        out_shape=jax.ShapeDtypeStruct((PT, H), jnp.float32),
        compiler_params=pltpu.CompilerParams(
            dimension_semantics=("arbitrary",),
            vmem_limit_bytes=100 * 1024 * 1024,
        ),
    )(block_expert, x_pad, w1, w3, w2)


def _sc_dispatch(xf, d0, d1, PT, CH=64):
    T, H = xf.shape
    per_w = T // NW
    nch = per_w // CH
    mesh = plsc.VectorSubcoreMesh(core_axis_name="c", subcore_axis_name="s")

    @functools.partial(
        pl.kernel,
        mesh=mesh,
        out_type=jax.ShapeDtypeStruct((PT, H), jnp.float32),
        scratch_types=[
            pltpu.VMEM((nch, CH), jnp.int32),
            pltpu.VMEM((nch, CH), jnp.int32),
            pltpu.VMEM((CH, H), jnp.float32),
            pltpu.SemaphoreType.DMA,
        ],
    )
    def k(x_hbm, d0_hbm, d1_hbm, out_hbm, d0_v, d1_v, rows_v, sem):
        wid = lax.axis_index("s") * NC + lax.axis_index("c")
        tbase = wid * per_w

        def body(c, carry):
            tb = tbase + c * CH
            pltpu.sync_copy(d0_hbm.at[pl.ds(tb, CH)], d0_v.at[c])
            pltpu.sync_copy(d1_hbm.at[pl.ds(tb, CH)], d1_v.at[c])
            pltpu.sync_copy(x_hbm.at[pl.ds(tb, CH)], rows_v)
            pltpu.async_copy(rows_v, out_hbm.at[d0_v.at[c]], sem)
            pltpu.async_copy(rows_v, out_hbm.at[d1_v.at[c]], sem)
            pltpu.make_async_copy(rows_v, out_hbm.at[d0_v.at[c]], sem).wait()
            pltpu.make_async_copy(rows_v, out_hbm.at[d1_v.at[c]], sem).wait()
            return carry

        lax.fori_loop(0, nch, body, 0)

    return k(xf, d0, d1)


def _lane_bcast(v16, j):
    idx = (jnp.zeros((16, 1), jnp.int32) + j).astype(jnp.int32)
    return lax.gather(
        v16, idx,
        lax.GatherDimensionNumbers(
            offset_dims=(), collapsed_slice_dims=(0,), start_index_map=(0,)
        ),
        slice_sizes=(1,),
        mode=lax.GatherScatterMode.PROMISE_IN_BOUNDS,
    )


def _sc_combine(y, d0, d1, w0, w1, CH=16):
    T = d0.shape[0]
    H = y.shape[1]
    per_w = T // NW
    nch = per_w // CH
    mesh = plsc.VectorSubcoreMesh(core_axis_name="c", subcore_axis_name="s")

    @functools.partial(
        pl.kernel,
        mesh=mesh,
        out_type=jax.ShapeDtypeStruct((T, H), jnp.float32),
        scratch_types=[
            pltpu.VMEM((per_w,), jnp.int32),
            pltpu.VMEM((per_w,), jnp.int32),
            pltpu.VMEM((per_w,), jnp.float32),
            pltpu.VMEM((per_w,), jnp.float32),
            [pltpu.VMEM((CH, H), jnp.float32) for _ in range(2)],
            [pltpu.VMEM((CH, H), jnp.float32) for _ in range(2)],
            [pltpu.SemaphoreType.DMA for _ in range(2)],
        ],
    )
    def k(y_hbm, d0_hbm, d1_hbm, w0_hbm, w1_hbm, out_hbm,
          i0_v, i1_v, w0_v, w1_v, r0_v, r1_v, sems):
        wid = lax.axis_index("s") * NC + lax.axis_index("c")
        base = wid * per_w
        pltpu.sync_copy(d0_hbm.at[pl.ds(base, per_w)], i0_v)
        pltpu.sync_copy(d1_hbm.at[pl.ds(base, per_w)], i1_v)
        pltpu.sync_copy(w0_hbm.at[pl.ds(base, per_w)], w0_v)
        pltpu.sync_copy(w1_hbm.at[pl.ds(base, per_w)], w1_v)

        def launch(c, sl):
            cs = pl.ds(c * CH, CH)
            pltpu.async_copy(y_hbm.at[i0_v.at[cs]], r0_v[sl], sems[sl])
            pltpu.async_copy(y_hbm.at[i1_v.at[cs]], r1_v[sl], sems[sl])

        launch(0, 0)
        for c in range(nch):
            sl = c & 1
            if c + 1 < nch:
                launch(c + 1, sl ^ 1)
            cs = pl.ds(c * CH, CH)
            pltpu.make_async_copy(y_hbm.at[i0_v.at[cs]], r0_v[sl], sems[sl]).wait()
            pltpu.make_async_copy(y_hbm.at[i1_v.at[cs]], r1_v[sl], sems[sl]).wait()
            wq0 = w0_v[pl.ds(c * CH, 16)]
            wq1 = w1_v[pl.ds(c * CH, 16)]

            def wrow(r16, cc, sl=sl, wq0=wq0, wq1=wq1):
                b0 = _lane_bcast(wq0, r16)
                b1 = _lane_bcast(wq1, r16)
                for col in range(H // 16):
                    csl = pl.ds(col * 16, 16)
                    r0_v[sl][r16, csl] = (
                        b0 * r0_v[sl][r16, csl] + b1 * r1_v[sl][r16, csl]
                    )
                return cc

            lax.fori_loop(0, CH, wrow, 0)
            pltpu.sync_copy(r0_v[sl], out_hbm.at[pl.ds(base + c * CH, CH)])

    return k(y, d0, d1, w0, w1)


def kernel(x, W_gate, w1, w3, w2):
    b, s, H = x.shape
    T = b * s
    E = W_gate.shape[0]
    xf = x.reshape(T, H)

    e0, e1, w0v, w1v, stats = _router(xf, W_gate.T)
    ssum = jnp.sum(stats, axis=(0, 1))
    probs_sum = ssum[:E]
    counts = ssum[E:2 * E]
    zsum = ssum[2 * E]
    balance_loss = E * jnp.sum((counts / (T * K)) * (probs_sum / T))
    z_loss = zsum / T

    PT = T * K + E * BM
    e_flat = jnp.concatenate([e0, e1])
    onehot = (e_flat[:, None] == jnp.arange(E)[None, :]).astype(jnp.int32)
    csum = jnp.cumsum(onehot, axis=0)
    rank = jnp.sum((csum - 1) * onehot, axis=1)
    g = csum[-1]
    padded = ((g + BM - 1) // BM) * BM
    pend = jnp.cumsum(padded).astype(jnp.int32)
    pstart = pend - padded.astype(jnp.int32)
    dest = jnp.sum(onehot * pstart[None, :], axis=1) + rank
    nb = PT // BM
    bstart = jnp.arange(nb, dtype=jnp.int32) * BM
    block_expert = jnp.minimum(
        jnp.sum((bstart[:, None] >= pend[None, :]).astype(jnp.int32), axis=1),
        E - 1,
    ).astype(jnp.int32)
    d0 = dest[:T]
    d1 = dest[T:]

    x_pad = _sc_dispatch(xf, d0, d1, PT)

    y = _ffn(block_expert, x_pad, w1, w3, w2)

    out = _sc_combine(y, d0, d1, w0v, w1v)

    return out.reshape(b, s, H), balance_loss, z_loss
